# baseline (speedup 1.0000x reference)
import jax
import jax.numpy as jnp
from jax import lax
from jax.experimental import pallas as pl
from jax.experimental.pallas import tpu as pltpu
from jax.experimental.pallas import tpu_sc as plsc

f32 = jnp.float32
i32 = jnp.int32
u32 = jnp.uint32

_NC = 2
_NS = 16
_NW = _NC * _NS
_C = 80
_BE = 512
_TOPK = 1024
_AUXW = 32


def _leaky(x):
    return jnp.where(x >= 0, x, 0.01 * x)


_LAG = 8


def _sc_bincount_body(zeros_hbm, row3_hbm, col3_hbm, cnt_out,
                      shared, rowt, colt, ones_v, sem):
    cid = lax.axis_index("c")
    sid = lax.axis_index("s")
    wid = sid * _NC + cid
    nch = row3_hbm.shape[1]

    @pl.when(sid == 0)
    def _init():
        pltpu.sync_copy(zeros_hbm, shared)

    pltpu.sync_copy(row3_hbm.at[wid], rowt)
    pltpu.sync_copy(col3_hbm.at[wid], colt)

    def _ones(i, c):
        ones_v[pl.ds(i * 16, 16)] = jnp.full((16,), 1.0, f32)
        return c
    lax.fori_loop(0, _C // 16, _ones, 0)
    plsc.subcore_barrier()

    def _drain2():
        pltpu.make_async_copy(ones_v, shared.at[rowt.at[0]], sem).wait()
        pltpu.make_async_copy(ones_v, shared.at[rowt.at[0]], sem).wait()

    def _body(j, c):
        pltpu.async_copy(ones_v, shared.at[rowt.at[j]], sem, add=True)
        pltpu.async_copy(ones_v, shared.at[colt.at[j]], sem, add=True)

        @pl.when(j >= _LAG)
        def _():
            _drain2()
        return c
    lax.fori_loop(0, nch, _body, 0)
    for _ in range(_LAG):
        _drain2()
    plsc.subcore_barrier()

    @pl.when(sid == 0)
    def _flush():
        pltpu.sync_copy(shared, cnt_out.at[cid])


def _sc_gather_body(h_hbm, pk_hbm, row3_hbm, col3_hbm,
                    sim_out, pkr_out, pkc_out,
                    rowt, colt, abuf, bbuf, prbuf, pcbuf, ptab,
                    gsem0, gsem1, wsem0, wsem1):
    cid = lax.axis_index("c")
    sid = lax.axis_index("s")
    wid = sid * _NC + cid
    E = sim_out.shape[0]
    epw = E // _NW
    base0 = wid * epw
    nch = row3_hbm.shape[1]
    gsems = (gsem0, gsem1)
    wsems = (wsem0, wsem1)

    pltpu.sync_copy(pk_hbm, ptab)
    pltpu.sync_copy(row3_hbm.at[wid], rowt)
    pltpu.sync_copy(col3_hbm.at[wid], colt)

    def fire_gather(j, b):
        pltpu.async_copy(h_hbm.at[rowt.at[j]], abuf.at[b], gsems[b])
        pltpu.async_copy(h_hbm.at[colt.at[j]], bbuf.at[b], gsems[b])

    def wait_gather(b):
        pltpu.make_async_copy(h_hbm.at[rowt.at[0]], abuf.at[b], gsems[b]).wait()
        pltpu.make_async_copy(h_hbm.at[colt.at[0]], bbuf.at[b], gsems[b]).wait()

    def compute(j, b):
        def _edge(e, c2):
            for s in range(8):
                a = abuf[b, e, pl.ds(s * 16, 16)]
                bv = bbuf[b, e, pl.ds(s * 16, 16)]
                abuf[b, e, pl.ds(s * 16, 16)] = jnp.exp(-jnp.abs(a - bv))
            return c2
        lax.fori_loop(0, _C, _edge, 0, unroll=2)
        for g in range(_C // 16):
            ir = rowt[j, pl.ds(g * 16, 16)]
            ic = colt[j, pl.ds(g * 16, 16)]
            prbuf[b, pl.ds(g * 16, 16)] = plsc.load_gather(ptab, [ir])
            pcbuf[b, pl.ds(g * 16, 16)] = plsc.load_gather(ptab, [ic])

    def fire_write(j, b):
        base = base0 + j * _C
        pltpu.async_copy(abuf.at[b], sim_out.at[pl.ds(base, _C)], wsems[b])
        pltpu.async_copy(prbuf.at[b], pkr_out.at[pl.ds(base, _C)], wsems[b])
        pltpu.async_copy(pcbuf.at[b], pkc_out.at[pl.ds(base, _C)], wsems[b])

    def drain_write(b):
        pltpu.make_async_copy(abuf.at[b], sim_out.at[pl.ds(0, _C)], wsems[b]).wait()
        pltpu.make_async_copy(prbuf.at[b], pkr_out.at[pl.ds(0, _C)], wsems[b]).wait()
        pltpu.make_async_copy(pcbuf.at[b], pkc_out.at[pl.ds(0, _C)], wsems[b]).wait()

    fire_gather(0, 0)
    fire_gather(1, 1)

    def _pair(i, c):
        j0 = 2 * i
        wait_gather(0)
        compute(j0, 0)
        fire_write(j0, 0)
        wait_gather(1)
        compute(j0 + 1, 1)
        fire_write(j0 + 1, 1)

        @pl.when(j0 + 2 < nch)
        def _():
            drain_write(0)
            fire_gather(j0 + 2, 0)

        @pl.when(j0 + 3 < nch)
        def _():
            drain_write(1)
            fire_gather(j0 + 3, 1)
        return c
    lax.fori_loop(0, nch // 2, _pair, 0)
    wait_gather(0)
    compute(nch - 1, 0)
    fire_write(nch - 1, 0)
    drain_write(0)
    drain_write(1)


def _prep_body(c0_ref, c1_ref, roles_ref, pk_ref):
    deg = jnp.log1p(c0_ref[...] + c1_ref[...])
    r = jnp.clip(roles_ref[...], 0, 2).astype(f32)
    pk_ref[...] = r * 16.0 + deg


def _pass1_body(sim_ref, pkr_ref, pkc_ref, rel_ref, iso_ref,
                w0s_ref, w0r_ref, b0_ref,
                w1s_ref, w1r_ref, wd_ref, rt_ref, w1rr_ref, w1rc_ref,
                w1o_ref, b1_ref,
                lng_ref, lnb_ref, w2_ref, b2_ref, w3_ref, b3_ref,
                m_ref, scal_ref, key_ref, stats_ref):
    it = pl.program_id(0)
    s = sim_ref[...]
    rl = rel_ref[...]
    io = iso_ref[...]

    m = (jnp.dot(s, w0s_ref[...], preferred_element_type=f32)
         + jnp.dot(rl, w0r_ref[...], preferred_element_type=f32)
         + b0_ref[...][0:1, :])
    m_ref[...] = m

    ps = jnp.concatenate(
        [jnp.sum(m, axis=0, keepdims=True),
         jnp.sum(m * m, axis=0, keepdims=True),
         jnp.zeros((6, m.shape[1]), f32)], axis=0)

    @pl.when(it == 0)
    def _():
        stats_ref[...] = ps

    @pl.when(it != 0)
    def _():
        stats_ref[...] = stats_ref[...] + ps

    pkr = pkr_ref[...]
    pkc = pkc_ref[...]
    rr = jnp.floor(pkr * (1.0 / 16.0))
    rc = jnp.floor(pkc * (1.0 / 16.0))
    dr = pkr - 16.0 * rr
    dc = pkc - 16.0 * rc
    lanes8 = lax.broadcasted_iota(i32, (1, 8), 1).astype(f32)
    ohr = (rr == lanes8).astype(f32)
    ohc = (rc == lanes8).astype(f32)
    rfr = jnp.dot(ohr, rt_ref[...], preferred_element_type=f32)
    rfc = jnp.dot(ohc, rt_ref[...], preferred_element_type=f32)
    b16 = lambda x: x.astype(jnp.bfloat16).astype(f32)
    hid = (jnp.dot(s, w1s_ref[...], preferred_element_type=f32)
           + jnp.dot(rl, w1r_ref[...], preferred_element_type=f32)
           + b16(dr) * b16(wd_ref[...][0:1, :])
           + b16(dc) * b16(wd_ref[...][1:2, :])
           + jnp.dot(rfr, w1rr_ref[...], preferred_element_type=f32)
           + jnp.dot(rfc, w1rc_ref[...], preferred_element_type=f32)
           + io * b16(w1o_ref[...][0:1, :])
           + b1_ref[...][0:1, :])
    mu = jnp.mean(hid, axis=1, keepdims=True)
    xc = hid - mu
    var = jnp.mean(xc * xc, axis=1, keepdims=True)
    hid = xc * lax.rsqrt(var + 1e-5) * lng_ref[...][0:1, :] + lnb_ref[...][0:1, :]
    hid = _leaky(hid)
    hid = _leaky(jnp.dot(hid, w2_ref[...], preferred_element_type=f32)
                 + b2_ref[...][0:1, :])
    g3 = jax.nn.sigmoid(jnp.dot(hid, w3_ref[...], preferred_element_type=f32)
                        + b3_ref[...][0:1, :])
    comp = g3[:, 2:3]
    scal_ref[...] = jnp.concatenate(
        [g3[:, 0:3], io, jnp.zeros((io.shape[0], 4), f32)], axis=1)

    masked = jnp.where(io > 0.5, -jnp.inf, comp)
    ub = lax.bitcast_convert_type(masked, u32)
    key = jnp.where(ub >= jnp.uint32(0x80000000),
                    jnp.bitwise_xor(ub, jnp.uint32(0xFFFFFFFF)),
                    jnp.bitwise_or(ub, jnp.uint32(0x80000000)))
    key_ref[...] = key


def _kth_body(keyw_ref, kth_ref):
    karr = keyw_ref[...]
    c = jnp.uint32(0)
    for b in range(32):
        t = jnp.bitwise_or(c, jnp.uint32(1 << (31 - b)))
        cnt = jnp.sum(jnp.where(karr >= t, 1.0, 0.0))
        c = jnp.where(cnt >= float(_TOPK), t, c)
    kth_ref[0, 0] = c


def _pass2_body(m_ref, scal_ref, key_ref, kth_ref, bs_ref,
                a_ref, b_ref, ws_ref, out_ref):
    mm = m_ref[...] * a_ref[...][0:1, :] + b_ref[...][0:1, :]
    x = _leaky(mm)
    ss = jnp.dot(x, ws_ref[...], preferred_element_type=f32) + bs_ref[0, 0]
    gate = scal_ref[:, 0:1]
    den = scal_ref[:, 1:2]
    comp = scal_ref[:, 2:3]
    io = scal_ref[:, 3:4]
    base = jax.nn.sigmoid(ss) * gate
    keep = (key_ref[...] >= kth_ref[0, 0]).astype(f32)
    out_ref[...] = base * jnp.where(io > 0.5, den, keep * comp)


def kernel(h, rel_embedding, params, row, col, is_original_edge, node_roles):
    N, EMB = h.shape
    E = row.shape[0]
    HID = params['enc_W2'].shape[0]

    row = row.astype(i32)
    col = col.astype(i32)
    iso = is_original_edge.astype(f32).reshape(E, 1)
    roles = node_roles.astype(i32).reshape(N, 1)
    epw = E // _NW
    nch = epw // _C
    row3 = row.reshape(_NW, nch, _C)
    col3 = col.reshape(_NW, nch, _C)

    mesh = plsc.VectorSubcoreMesh(core_axis_name="c", subcore_axis_name="s")

    bincount = pl.kernel(
        _sc_bincount_body,
        out_type=jax.ShapeDtypeStruct((_NC, N), f32),
        mesh=mesh,
        scratch_types=[
            pltpu.VMEM_SHARED((N,), f32),
            pltpu.VMEM((nch, _C), i32),
            pltpu.VMEM((nch, _C), i32),
            pltpu.VMEM((_C,), f32),
            pltpu.SemaphoreType.DMA,
        ],
    )
    cnts = bincount(jnp.zeros((N,), f32), row3, col3)
    c0 = cnts[0].reshape(N, 1)
    c1 = cnts[1].reshape(N, 1)

    packed = pl.pallas_call(
        _prep_body,
        out_shape=jax.ShapeDtypeStruct((N, 1), f32),
    )(c0, c1, roles).reshape(N)

    gat = pl.kernel(
        _sc_gather_body,
        out_type=[
            jax.ShapeDtypeStruct((E, EMB), f32),
            jax.ShapeDtypeStruct((E,), f32),
            jax.ShapeDtypeStruct((E,), f32),
        ],
        mesh=mesh,
        scratch_types=[
            pltpu.VMEM((nch, _C), i32),
            pltpu.VMEM((nch, _C), i32),
            pltpu.VMEM((2, _C, EMB), f32),
            pltpu.VMEM((2, _C, EMB), f32),
            pltpu.VMEM((2, _C), f32),
            pltpu.VMEM((2, _C), f32),
            pltpu.VMEM((N,), f32),
            pltpu.SemaphoreType.DMA,
            pltpu.SemaphoreType.DMA,
            pltpu.SemaphoreType.DMA,
            pltpu.SemaphoreType.DMA,
        ],
        compiler_params=pltpu.CompilerParams(needs_layout_passes=False),
    )
    sim, pkr, pkc = gat(h, packed, row3, col3)
    pkr = pkr.reshape(E, 1)
    pkc = pkc.reshape(E, 1)

    W1 = params['enc_W1']
    w1s = W1[0:EMB]
    w1r = W1[EMB:EMB + 16]
    wd = jnp.zeros((8, HID), f32).at[0].set(W1[EMB + 17]) \
        .at[1].set(W1[EMB + 18])
    rt8 = jnp.zeros((8, 16), f32).at[0:3].set(params['role_table'])
    w1rr = W1[EMB + 19:EMB + 35]
    w1rc = W1[EMB + 35:EMB + 51]
    w1o = jnp.zeros((8, HID), f32).at[0].set(W1[EMB + 16])

    def pad8(v):
        return jnp.zeros((8, v.shape[-1]), f32).at[0].set(v.reshape(-1))

    w3 = jnp.zeros((HID, 8), f32) \
        .at[:, 0].set(params['gate_W'][:, 0]) \
        .at[:, 1].set(params['den_W'][:, 0]) \
        .at[:, 2].set(params['comp_W'][:, 0])
    b3 = jnp.zeros((8, 8), f32) \
        .at[0, 0].set(params['gate_b'][0]) \
        .at[0, 1].set(params['den_b'][0]) \
        .at[0, 2].set(params['comp_b'][0])

    nblk = E // _BE
    full = lambda arr: pl.BlockSpec(arr.shape, lambda i: (0,) * arr.ndim)

    def ebs(w):
        return pl.BlockSpec((_BE, w), lambda i: (i, 0))

    wspecs = []
    wargs = [params['mlp_W0'][0:EMB], params['mlp_W0'][EMB:EMB + 16],
             pad8(params['mlp_b0']),
             w1s, w1r, wd, rt8, w1rr, w1rc, w1o, pad8(params['enc_b1']),
             pad8(params['ln_g']), pad8(params['ln_b']),
             params['enc_W2'], pad8(params['enc_b2']), w3, b3]
    for a in wargs:
        wspecs.append(full(a))

    m_arr, scal, key, stats = pl.pallas_call(
        _pass1_body,
        grid=(nblk,),
        in_specs=[ebs(EMB), ebs(1), ebs(1), ebs(16), ebs(1)] + wspecs,
        out_specs=[ebs(HID), ebs(8), ebs(1),
                   pl.BlockSpec((8, HID), lambda i: (0, 0))],
        out_shape=[jax.ShapeDtypeStruct((E, HID), f32),
                   jax.ShapeDtypeStruct((E, 8), f32),
                   jax.ShapeDtypeStruct((E, 1), u32),
                   jax.ShapeDtypeStruct((8, HID), f32)],
    )(sim, pkr, pkc, rel_embedding, iso, *wargs)

    keyw = key.reshape(E // 128, 128)
    kth = pl.pallas_call(
        _kth_body,
        out_shape=jax.ShapeDtypeStruct((1, 1), u32),
        out_specs=pl.BlockSpec(memory_space=pltpu.SMEM),
    )(keyw)

    mu = stats[0] / float(E)
    ex2 = stats[1] / float(E)
    var = ex2 - mu * mu
    a_vec = params['bn_g'] * lax.rsqrt(var + 1e-5)
    b_vec = params['bn_b'] - mu * a_vec
    bs = params['mlp_bs'].reshape(1, 1)

    weight = pl.pallas_call(
        _pass2_body,
        grid=(nblk,),
        in_specs=[ebs(HID), ebs(8), ebs(1),
                  pl.BlockSpec(memory_space=pltpu.SMEM),
                  pl.BlockSpec(memory_space=pltpu.SMEM),
                  full(jnp.zeros((8, HID))), full(jnp.zeros((8, HID))),
                  pl.BlockSpec((HID, 1), lambda i: (0, 0))],
        out_specs=ebs(1),
        out_shape=jax.ShapeDtypeStruct((E, 1), f32),
    )(m_arr, scal, key, kth, bs, pad8(a_vec), pad8(b_vec),
      params['mlp_Ws'])

    return weight


# speedup vs baseline: 1.2252x; 1.2252x over previous
import jax
import jax.numpy as jnp
from jax import lax
from jax.experimental import pallas as pl
from jax.experimental.pallas import tpu as pltpu
from jax.experimental.pallas import tpu_sc as plsc

f32 = jnp.float32
i32 = jnp.int32
u32 = jnp.uint32

_NC = 2
_NS = 16
_NW = _NC * _NS
_C = 80
_BE = 512
_TOPK = 1024
_AUXW = 32


def _leaky(x):
    return jnp.where(x >= 0, x, 0.01 * x)


_LAG = 8


def _sc_bincount_body(zeros_hbm, row3_hbm, col3_hbm, cnt_out,
                      shared, rowt, colt, ones_v, sem):
    cid = lax.axis_index("c")
    sid = lax.axis_index("s")
    wid = sid * _NC + cid
    nch = row3_hbm.shape[1]

    @pl.when(sid == 0)
    def _init():
        pltpu.sync_copy(zeros_hbm, shared)

    pltpu.sync_copy(row3_hbm.at[wid], rowt)
    pltpu.sync_copy(col3_hbm.at[wid], colt)

    def _ones(i, c):
        ones_v[pl.ds(i * 16, 16)] = jnp.full((16,), 1.0, f32)
        return c
    lax.fori_loop(0, _C // 16, _ones, 0)
    plsc.subcore_barrier()

    def _drain2():
        pltpu.make_async_copy(ones_v, shared.at[rowt.at[0]], sem).wait()
        pltpu.make_async_copy(ones_v, shared.at[rowt.at[0]], sem).wait()

    def _body(j, c):
        pltpu.async_copy(ones_v, shared.at[rowt.at[j]], sem, add=True)
        pltpu.async_copy(ones_v, shared.at[colt.at[j]], sem, add=True)

        @pl.when(j >= _LAG)
        def _():
            _drain2()
        return c
    lax.fori_loop(0, nch, _body, 0)
    for _ in range(_LAG):
        _drain2()
    plsc.subcore_barrier()

    @pl.when(sid == 0)
    def _flush():
        pltpu.sync_copy(shared, cnt_out.at[cid])


def _sc_gather_body(h_hbm, pk_hbm, row3_hbm, col3_hbm,
                    sim_out, pkr_out, pkc_out,
                    rowt, colt, abuf, bbuf, prbuf, pcbuf, ptab,
                    gsem0, gsem1, gsem2, gsem3, wsem0, wsem1, wsem2, wsem3):
    cid = lax.axis_index("c")
    sid = lax.axis_index("s")
    wid = sid * _NC + cid
    E = sim_out.shape[0]
    epw = E // _NW
    base0 = wid * epw
    nch = row3_hbm.shape[1]
    gsems = (gsem0, gsem1, gsem2, gsem3)
    wsems = (wsem0, wsem1, wsem2, wsem3)

    pltpu.sync_copy(pk_hbm, ptab)
    pltpu.sync_copy(row3_hbm.at[wid], rowt)
    pltpu.sync_copy(col3_hbm.at[wid], colt)

    def fire_gather(j, b):
        pltpu.async_copy(h_hbm.at[rowt.at[j]], abuf.at[b], gsems[b])
        pltpu.async_copy(h_hbm.at[colt.at[j]], bbuf.at[b], gsems[b])

    def wait_gather(b):
        pltpu.make_async_copy(h_hbm.at[pl.ds(0, _C)], abuf.at[b], gsems[b]).wait()
        pltpu.make_async_copy(h_hbm.at[pl.ds(0, _C)], bbuf.at[b], gsems[b]).wait()

    def compute(j, b):
        def _edge(e, c2):
            for s in range(8):
                a = abuf[b, e, pl.ds(s * 16, 16)]
                bv = bbuf[b, e, pl.ds(s * 16, 16)]
                abuf[b, e, pl.ds(s * 16, 16)] = jnp.exp(-jnp.abs(a - bv))
            return c2
        lax.fori_loop(0, _C, _edge, 0, unroll=2)
        for g in range(_C // 16):
            ir = rowt[j, pl.ds(g * 16, 16)]
            ic = colt[j, pl.ds(g * 16, 16)]
            prbuf[b, pl.ds(g * 16, 16)] = plsc.load_gather(ptab, [ir])
            pcbuf[b, pl.ds(g * 16, 16)] = plsc.load_gather(ptab, [ic])

    def fire_write(j, b):
        base = base0 + j * _C
        pltpu.async_copy(abuf.at[b], sim_out.at[pl.ds(base, _C)], wsems[b])
        pltpu.async_copy(prbuf.at[b], pkr_out.at[pl.ds(base, _C)], wsems[b])
        pltpu.async_copy(pcbuf.at[b], pkc_out.at[pl.ds(base, _C)], wsems[b])

    def drain_write(b):
        pltpu.make_async_copy(abuf.at[b], sim_out.at[pl.ds(0, _C)], wsems[b]).wait()
        pltpu.make_async_copy(prbuf.at[b], pkr_out.at[pl.ds(0, _C)], wsems[b]).wait()
        pltpu.make_async_copy(pcbuf.at[b], pkc_out.at[pl.ds(0, _C)], wsems[b]).wait()

    fire_gather(0, 0)
    fire_gather(1, 1)
    fire_gather(2, 2)

    def _step(j, s):
        wait_gather(s)
        compute(j, s)
        fire_write(j, s)
        sp = (s + 3) % 4

        @pl.when(j >= 1)
        def _():
            drain_write(sp)

        @pl.when(j + 3 < nch)
        def _():
            fire_gather(j + 3, sp)

    def _quad(i, c):
        j0 = 4 * i
        for k in range(4):
            _step(j0 + k, k)
        return c
    lax.fori_loop(0, (nch - 1) // 4, _quad, 0)
    wait_gather(0)
    compute(nch - 1, 0)
    fire_write(nch - 1, 0)
    drain_write(3)
    drain_write(0)


def _prep_body(c0_ref, c1_ref, roles_ref, pk_ref):
    deg = jnp.log1p(c0_ref[...] + c1_ref[...])
    r = jnp.clip(roles_ref[...], 0, 2).astype(f32)
    pk_ref[...] = r * 16.0 + deg


def _pass1_body(sim_ref, pkr_ref, pkc_ref, rel_ref, iso_ref,
                w0s_ref, w0r_ref, b0_ref,
                w1s_ref, w1r_ref, wd_ref, rt_ref, w1rr_ref, w1rc_ref,
                w1o_ref, b1_ref,
                lng_ref, lnb_ref, w2_ref, b2_ref, w3_ref, b3_ref,
                m_ref, scal_ref, key_ref, stats_ref):
    it = pl.program_id(0)
    s = sim_ref[...]
    rl = rel_ref[...]
    io = iso_ref[...]

    m = (jnp.dot(s, w0s_ref[...], preferred_element_type=f32)
         + jnp.dot(rl, w0r_ref[...], preferred_element_type=f32)
         + b0_ref[...][0:1, :])
    m_ref[...] = m

    ps = jnp.concatenate(
        [jnp.sum(m, axis=0, keepdims=True),
         jnp.sum(m * m, axis=0, keepdims=True),
         jnp.zeros((6, m.shape[1]), f32)], axis=0)

    @pl.when(it == 0)
    def _():
        stats_ref[...] = ps

    @pl.when(it != 0)
    def _():
        stats_ref[...] = stats_ref[...] + ps

    pkr = pkr_ref[...]
    pkc = pkc_ref[...]
    rr = jnp.floor(pkr * (1.0 / 16.0))
    rc = jnp.floor(pkc * (1.0 / 16.0))
    dr = pkr - 16.0 * rr
    dc = pkc - 16.0 * rc
    lanes8 = lax.broadcasted_iota(i32, (1, 8), 1).astype(f32)
    ohr = (rr == lanes8).astype(f32)
    ohc = (rc == lanes8).astype(f32)
    rfr = jnp.dot(ohr, rt_ref[...], preferred_element_type=f32)
    rfc = jnp.dot(ohc, rt_ref[...], preferred_element_type=f32)
    b16 = lambda x: x.astype(jnp.bfloat16).astype(f32)
    hid = (jnp.dot(s, w1s_ref[...], preferred_element_type=f32)
           + jnp.dot(rl, w1r_ref[...], preferred_element_type=f32)
           + b16(dr) * b16(wd_ref[...][0:1, :])
           + b16(dc) * b16(wd_ref[...][1:2, :])
           + jnp.dot(rfr, w1rr_ref[...], preferred_element_type=f32)
           + jnp.dot(rfc, w1rc_ref[...], preferred_element_type=f32)
           + io * b16(w1o_ref[...][0:1, :])
           + b1_ref[...][0:1, :])
    mu = jnp.mean(hid, axis=1, keepdims=True)
    xc = hid - mu
    var = jnp.mean(xc * xc, axis=1, keepdims=True)
    hid = xc * lax.rsqrt(var + 1e-5) * lng_ref[...][0:1, :] + lnb_ref[...][0:1, :]
    hid = _leaky(hid)
    hid = _leaky(jnp.dot(hid, w2_ref[...], preferred_element_type=f32)
                 + b2_ref[...][0:1, :])
    g3 = jax.nn.sigmoid(jnp.dot(hid, w3_ref[...], preferred_element_type=f32)
                        + b3_ref[...][0:1, :])
    comp = g3[:, 2:3]
    scal_ref[...] = jnp.concatenate(
        [g3[:, 0:3], io, jnp.zeros((io.shape[0], 4), f32)], axis=1)

    masked = jnp.where(io > 0.5, -jnp.inf, comp)
    ub = lax.bitcast_convert_type(masked, u32)
    key = jnp.where(ub >= jnp.uint32(0x80000000),
                    jnp.bitwise_xor(ub, jnp.uint32(0xFFFFFFFF)),
                    jnp.bitwise_or(ub, jnp.uint32(0x80000000)))
    key_ref[...] = key


def _kth_body(keyw_ref, kth_ref):
    karr = keyw_ref[...]
    c = jnp.uint32(0)
    for b in range(32):
        t = jnp.bitwise_or(c, jnp.uint32(1 << (31 - b)))
        cnt = jnp.sum(jnp.where(karr >= t, 1.0, 0.0))
        c = jnp.where(cnt >= float(_TOPK), t, c)
    kth_ref[0, 0] = c


def _pass2_body(m_ref, scal_ref, key_ref, kth_ref, bs_ref,
                a_ref, b_ref, ws_ref, out_ref):
    mm = m_ref[...] * a_ref[...][0:1, :] + b_ref[...][0:1, :]
    x = _leaky(mm)
    ss = jnp.dot(x, ws_ref[...], preferred_element_type=f32) + bs_ref[0, 0]
    gate = scal_ref[:, 0:1]
    den = scal_ref[:, 1:2]
    comp = scal_ref[:, 2:3]
    io = scal_ref[:, 3:4]
    base = jax.nn.sigmoid(ss) * gate
    keep = (key_ref[...] >= kth_ref[0, 0]).astype(f32)
    out_ref[...] = base * jnp.where(io > 0.5, den, keep * comp)


def kernel(h, rel_embedding, params, row, col, is_original_edge, node_roles):
    N, EMB = h.shape
    E = row.shape[0]
    HID = params['enc_W2'].shape[0]

    row = row.astype(i32)
    col = col.astype(i32)
    iso = is_original_edge.astype(f32).reshape(E, 1)
    roles = node_roles.astype(i32).reshape(N, 1)
    epw = E // _NW
    nch = epw // _C
    row3 = row.reshape(_NW, nch, _C)
    col3 = col.reshape(_NW, nch, _C)

    mesh = plsc.VectorSubcoreMesh(core_axis_name="c", subcore_axis_name="s")

    bincount = pl.kernel(
        _sc_bincount_body,
        out_type=jax.ShapeDtypeStruct((_NC, N), f32),
        mesh=mesh,
        scratch_types=[
            pltpu.VMEM_SHARED((N,), f32),
            pltpu.VMEM((nch, _C), i32),
            pltpu.VMEM((nch, _C), i32),
            pltpu.VMEM((_C,), f32),
            pltpu.SemaphoreType.DMA,
        ],
    )
    cnts = bincount(jnp.zeros((N,), f32), row3, col3)
    c0 = cnts[0].reshape(N, 1)
    c1 = cnts[1].reshape(N, 1)

    packed = pl.pallas_call(
        _prep_body,
        out_shape=jax.ShapeDtypeStruct((N, 1), f32),
    )(c0, c1, roles).reshape(N)

    gat = pl.kernel(
        _sc_gather_body,
        out_type=[
            jax.ShapeDtypeStruct((E, EMB), f32),
            jax.ShapeDtypeStruct((E,), f32),
            jax.ShapeDtypeStruct((E,), f32),
        ],
        mesh=mesh,
        scratch_types=[
            pltpu.VMEM((nch, _C), i32),
            pltpu.VMEM((nch, _C), i32),
            pltpu.VMEM((4, _C, EMB), f32),
            pltpu.VMEM((4, _C, EMB), f32),
            pltpu.VMEM((4, _C), f32),
            pltpu.VMEM((4, _C), f32),
            pltpu.VMEM((N,), f32),
        ] + [pltpu.SemaphoreType.DMA] * 8,
        compiler_params=pltpu.CompilerParams(needs_layout_passes=False),
    )
    sim, pkr, pkc = gat(h, packed, row3, col3)
    pkr = pkr.reshape(E, 1)
    pkc = pkc.reshape(E, 1)

    W1 = params['enc_W1']
    w1s = W1[0:EMB]
    w1r = W1[EMB:EMB + 16]
    wd = jnp.zeros((8, HID), f32).at[0].set(W1[EMB + 17]) \
        .at[1].set(W1[EMB + 18])
    rt8 = jnp.zeros((8, 16), f32).at[0:3].set(params['role_table'])
    w1rr = W1[EMB + 19:EMB + 35]
    w1rc = W1[EMB + 35:EMB + 51]
    w1o = jnp.zeros((8, HID), f32).at[0].set(W1[EMB + 16])

    def pad8(v):
        return jnp.zeros((8, v.shape[-1]), f32).at[0].set(v.reshape(-1))

    w3 = jnp.zeros((HID, 8), f32) \
        .at[:, 0].set(params['gate_W'][:, 0]) \
        .at[:, 1].set(params['den_W'][:, 0]) \
        .at[:, 2].set(params['comp_W'][:, 0])
    b3 = jnp.zeros((8, 8), f32) \
        .at[0, 0].set(params['gate_b'][0]) \
        .at[0, 1].set(params['den_b'][0]) \
        .at[0, 2].set(params['comp_b'][0])

    nblk = E // _BE
    full = lambda arr: pl.BlockSpec(arr.shape, lambda i: (0,) * arr.ndim)

    def ebs(w):
        return pl.BlockSpec((_BE, w), lambda i: (i, 0))

    wspecs = []
    wargs = [params['mlp_W0'][0:EMB], params['mlp_W0'][EMB:EMB + 16],
             pad8(params['mlp_b0']),
             w1s, w1r, wd, rt8, w1rr, w1rc, w1o, pad8(params['enc_b1']),
             pad8(params['ln_g']), pad8(params['ln_b']),
             params['enc_W2'], pad8(params['enc_b2']), w3, b3]
    for a in wargs:
        wspecs.append(full(a))

    m_arr, scal, key, stats = pl.pallas_call(
        _pass1_body,
        grid=(nblk,),
        in_specs=[ebs(EMB), ebs(1), ebs(1), ebs(16), ebs(1)] + wspecs,
        out_specs=[ebs(HID), ebs(8), ebs(1),
                   pl.BlockSpec((8, HID), lambda i: (0, 0))],
        out_shape=[jax.ShapeDtypeStruct((E, HID), f32),
                   jax.ShapeDtypeStruct((E, 8), f32),
                   jax.ShapeDtypeStruct((E, 1), u32),
                   jax.ShapeDtypeStruct((8, HID), f32)],
    )(sim, pkr, pkc, rel_embedding, iso, *wargs)

    keyw = key.reshape(E // 128, 128)
    kth = pl.pallas_call(
        _kth_body,
        out_shape=jax.ShapeDtypeStruct((1, 1), u32),
        out_specs=pl.BlockSpec(memory_space=pltpu.SMEM),
    )(keyw)

    mu = stats[0] / float(E)
    ex2 = stats[1] / float(E)
    var = ex2 - mu * mu
    a_vec = params['bn_g'] * lax.rsqrt(var + 1e-5)
    b_vec = params['bn_b'] - mu * a_vec
    bs = params['mlp_bs'].reshape(1, 1)

    weight = pl.pallas_call(
        _pass2_body,
        grid=(nblk,),
        in_specs=[ebs(HID), ebs(8), ebs(1),
                  pl.BlockSpec(memory_space=pltpu.SMEM),
                  pl.BlockSpec(memory_space=pltpu.SMEM),
                  full(jnp.zeros((8, HID))), full(jnp.zeros((8, HID))),
                  pl.BlockSpec((HID, 1), lambda i: (0, 0))],
        out_specs=ebs(1),
        out_shape=jax.ShapeDtypeStruct((E, 1), f32),
    )(m_arr, scal, key, kth, bs, pad8(a_vec), pad8(b_vec),
      params['mlp_Ws'])

    return weight


# speedup vs baseline: 1.3825x; 1.1284x over previous
import jax
import jax.numpy as jnp
from jax import lax
from jax.experimental import pallas as pl
from jax.experimental.pallas import tpu as pltpu
from jax.experimental.pallas import tpu_sc as plsc

f32 = jnp.float32
i32 = jnp.int32
u32 = jnp.uint32

_NC = 2
_NS = 16
_NW = _NC * _NS
_C = 80
_BE = 512
_TOPK = 1024
_AUXW = 32


def _leaky(x):
    return jnp.where(x >= 0, x, 0.01 * x)


_LAG = 8


def _sc_bincount_body(zeros_hbm, row3_hbm, col3_hbm, cnt_out,
                      shared, rowt, colt, ones_v, sem):
    cid = lax.axis_index("c")
    sid = lax.axis_index("s")
    wid = sid * _NC + cid
    nch = row3_hbm.shape[1]

    @pl.when(sid == 0)
    def _init():
        pltpu.sync_copy(zeros_hbm, shared)

    pltpu.sync_copy(row3_hbm.at[wid], rowt)
    pltpu.sync_copy(col3_hbm.at[wid], colt)

    def _ones(i, c):
        ones_v[pl.ds(i * 16, 16)] = jnp.full((16,), 1.0, f32)
        return c
    lax.fori_loop(0, _C // 16, _ones, 0)
    plsc.subcore_barrier()

    def _drain2():
        pltpu.make_async_copy(ones_v, shared.at[rowt.at[0]], sem).wait()
        pltpu.make_async_copy(ones_v, shared.at[rowt.at[0]], sem).wait()

    def _body(j, c):
        pltpu.async_copy(ones_v, shared.at[rowt.at[j]], sem, add=True)
        pltpu.async_copy(ones_v, shared.at[colt.at[j]], sem, add=True)

        @pl.when(j >= _LAG)
        def _():
            _drain2()
        return c
    lax.fori_loop(0, nch, _body, 0)
    for _ in range(_LAG):
        _drain2()
    plsc.subcore_barrier()

    @pl.when(sid == 0)
    def _flush():
        pltpu.sync_copy(shared, cnt_out.at[cid])


def _sc_gather_body(h_hbm, pk_hbm, row3_hbm, col3_hbm,
                    sim_out, pkr_out, pkc_out,
                    rowt, colt, abuf, bbuf, prbuf, pcbuf, ptab,
                    gsem0, gsem1, gsem2, gsem3, wsem0, wsem1, wsem2, wsem3):
    cid = lax.axis_index("c")
    sid = lax.axis_index("s")
    wid = sid * _NC + cid
    E = sim_out.shape[0]
    epw = E // _NW
    base0 = wid * epw
    nch = row3_hbm.shape[1]
    gsems = (gsem0, gsem1, gsem2, gsem3)
    wsems = (wsem0, wsem1, wsem2, wsem3)

    pltpu.sync_copy(pk_hbm, ptab)
    pltpu.sync_copy(row3_hbm.at[wid], rowt)
    pltpu.sync_copy(col3_hbm.at[wid], colt)

    def fire_gather(j, b):
        pltpu.async_copy(h_hbm.at[rowt.at[j]], abuf.at[b], gsems[b])
        pltpu.async_copy(h_hbm.at[colt.at[j]], bbuf.at[b], gsems[b])

    def wait_gather(b):
        pltpu.make_async_copy(h_hbm.at[pl.ds(0, _C)], abuf.at[b], gsems[b]).wait()
        pltpu.make_async_copy(h_hbm.at[pl.ds(0, _C)], bbuf.at[b], gsems[b]).wait()

    def compute(j, b):
        @plsc.parallel_loop(0, _C, step=1, unroll=4)
        def _edge(e):
            for s in range(8):
                a = abuf[b, e, pl.ds(s * 16, 16)]
                bv = bbuf[b, e, pl.ds(s * 16, 16)]
                abuf[b, e, pl.ds(s * 16, 16)] = jnp.exp(-jnp.abs(a - bv))
        for g in range(_C // 16):
            ir = rowt[j, pl.ds(g * 16, 16)]
            ic = colt[j, pl.ds(g * 16, 16)]
            prbuf[b, pl.ds(g * 16, 16)] = plsc.load_gather(ptab, [ir])
            pcbuf[b, pl.ds(g * 16, 16)] = plsc.load_gather(ptab, [ic])

    def fire_write(j, b):
        base = base0 + j * _C
        pltpu.async_copy(abuf.at[b], sim_out.at[pl.ds(base, _C)], wsems[b])
        pltpu.async_copy(prbuf.at[b], pkr_out.at[pl.ds(base, _C)], wsems[b])
        pltpu.async_copy(pcbuf.at[b], pkc_out.at[pl.ds(base, _C)], wsems[b])

    def drain_write(b):
        pltpu.make_async_copy(abuf.at[b], sim_out.at[pl.ds(0, _C)], wsems[b]).wait()
        pltpu.make_async_copy(prbuf.at[b], pkr_out.at[pl.ds(0, _C)], wsems[b]).wait()
        pltpu.make_async_copy(pcbuf.at[b], pkc_out.at[pl.ds(0, _C)], wsems[b]).wait()

    fire_gather(0, 0)
    fire_gather(1, 1)
    fire_gather(2, 2)

    def _step(j, s):
        wait_gather(s)
        compute(j, s)
        fire_write(j, s)
        sp = (s + 3) % 4

        @pl.when(j >= 1)
        def _():
            drain_write(sp)

        @pl.when(j + 3 < nch)
        def _():
            fire_gather(j + 3, sp)

    def _quad(i, c):
        j0 = 4 * i
        for k in range(4):
            _step(j0 + k, k)
        return c
    lax.fori_loop(0, (nch - 1) // 4, _quad, 0)
    wait_gather(0)
    compute(nch - 1, 0)
    fire_write(nch - 1, 0)
    drain_write(3)
    drain_write(0)


def _prep_body(c0_ref, c1_ref, roles_ref, pk_ref):
    deg = jnp.log1p(c0_ref[...] + c1_ref[...])
    r = jnp.clip(roles_ref[...], 0, 2).astype(f32)
    pk_ref[...] = r * 16.0 + deg


def _pass1_body(sim_ref, pkr_ref, pkc_ref, rel_ref, iso_ref,
                w0s_ref, w0r_ref, b0_ref,
                w1s_ref, w1r_ref, wd_ref, rt_ref, w1rr_ref, w1rc_ref,
                w1o_ref, b1_ref,
                lng_ref, lnb_ref, w2_ref, b2_ref, w3_ref, b3_ref,
                m_ref, scal_ref, key_ref, stats_ref):
    it = pl.program_id(0)
    s = sim_ref[...]
    rl = rel_ref[...]
    io = iso_ref[...]

    m = (jnp.dot(s, w0s_ref[...], preferred_element_type=f32)
         + jnp.dot(rl, w0r_ref[...], preferred_element_type=f32)
         + b0_ref[...][0:1, :])
    m_ref[...] = m

    ps = jnp.concatenate(
        [jnp.sum(m, axis=0, keepdims=True),
         jnp.sum(m * m, axis=0, keepdims=True),
         jnp.zeros((6, m.shape[1]), f32)], axis=0)

    @pl.when(it == 0)
    def _():
        stats_ref[...] = ps

    @pl.when(it != 0)
    def _():
        stats_ref[...] = stats_ref[...] + ps

    pkr = pkr_ref[...]
    pkc = pkc_ref[...]
    rr = jnp.floor(pkr * (1.0 / 16.0))
    rc = jnp.floor(pkc * (1.0 / 16.0))
    dr = pkr - 16.0 * rr
    dc = pkc - 16.0 * rc
    lanes8 = lax.broadcasted_iota(i32, (1, 8), 1).astype(f32)
    ohr = (rr == lanes8).astype(f32)
    ohc = (rc == lanes8).astype(f32)
    rfr = jnp.dot(ohr, rt_ref[...], preferred_element_type=f32)
    rfc = jnp.dot(ohc, rt_ref[...], preferred_element_type=f32)
    b16 = lambda x: x.astype(jnp.bfloat16).astype(f32)
    hid = (jnp.dot(s, w1s_ref[...], preferred_element_type=f32)
           + jnp.dot(rl, w1r_ref[...], preferred_element_type=f32)
           + b16(dr) * b16(wd_ref[...][0:1, :])
           + b16(dc) * b16(wd_ref[...][1:2, :])
           + jnp.dot(rfr, w1rr_ref[...], preferred_element_type=f32)
           + jnp.dot(rfc, w1rc_ref[...], preferred_element_type=f32)
           + io * b16(w1o_ref[...][0:1, :])
           + b1_ref[...][0:1, :])
    mu = jnp.mean(hid, axis=1, keepdims=True)
    xc = hid - mu
    var = jnp.mean(xc * xc, axis=1, keepdims=True)
    hid = xc * lax.rsqrt(var + 1e-5) * lng_ref[...][0:1, :] + lnb_ref[...][0:1, :]
    hid = _leaky(hid)
    hid = _leaky(jnp.dot(hid, w2_ref[...], preferred_element_type=f32)
                 + b2_ref[...][0:1, :])
    g3 = jax.nn.sigmoid(jnp.dot(hid, w3_ref[...], preferred_element_type=f32)
                        + b3_ref[...][0:1, :])
    comp = g3[:, 2:3]
    scal_ref[...] = jnp.concatenate(
        [g3[:, 0:3], io, jnp.zeros((io.shape[0], 4), f32)], axis=1)

    masked = jnp.where(io > 0.5, -jnp.inf, comp)
    ub = lax.bitcast_convert_type(masked, u32)
    key = jnp.where(ub >= jnp.uint32(0x80000000),
                    jnp.bitwise_xor(ub, jnp.uint32(0xFFFFFFFF)),
                    jnp.bitwise_or(ub, jnp.uint32(0x80000000)))
    key_ref[...] = key


def _kth_body(keyw_ref, kth_ref):
    karr = keyw_ref[...]
    c = jnp.uint32(0)
    for b in range(32):
        t = jnp.bitwise_or(c, jnp.uint32(1 << (31 - b)))
        cnt = jnp.sum(jnp.where(karr >= t, 1.0, 0.0))
        c = jnp.where(cnt >= float(_TOPK), t, c)
    kth_ref[0, 0] = c


def _pass2_body(m_ref, scal_ref, key_ref, kth_ref, bs_ref,
                a_ref, b_ref, ws_ref, out_ref):
    mm = m_ref[...] * a_ref[...][0:1, :] + b_ref[...][0:1, :]
    x = _leaky(mm)
    ss = jnp.dot(x, ws_ref[...], preferred_element_type=f32) + bs_ref[0, 0]
    gate = scal_ref[:, 0:1]
    den = scal_ref[:, 1:2]
    comp = scal_ref[:, 2:3]
    io = scal_ref[:, 3:4]
    base = jax.nn.sigmoid(ss) * gate
    keep = (key_ref[...] >= kth_ref[0, 0]).astype(f32)
    out_ref[...] = base * jnp.where(io > 0.5, den, keep * comp)


def kernel(h, rel_embedding, params, row, col, is_original_edge, node_roles):
    N, EMB = h.shape
    E = row.shape[0]
    HID = params['enc_W2'].shape[0]

    row = row.astype(i32)
    col = col.astype(i32)
    iso = is_original_edge.astype(f32).reshape(E, 1)
    roles = node_roles.astype(i32).reshape(N, 1)
    epw = E // _NW
    nch = epw // _C
    row3 = row.reshape(_NW, nch, _C)
    col3 = col.reshape(_NW, nch, _C)

    mesh = plsc.VectorSubcoreMesh(core_axis_name="c", subcore_axis_name="s")

    bincount = pl.kernel(
        _sc_bincount_body,
        out_type=jax.ShapeDtypeStruct((_NC, N), f32),
        mesh=mesh,
        scratch_types=[
            pltpu.VMEM_SHARED((N,), f32),
            pltpu.VMEM((nch, _C), i32),
            pltpu.VMEM((nch, _C), i32),
            pltpu.VMEM((_C,), f32),
            pltpu.SemaphoreType.DMA,
        ],
    )
    cnts = bincount(jnp.zeros((N,), f32), row3, col3)
    c0 = cnts[0].reshape(N, 1)
    c1 = cnts[1].reshape(N, 1)

    packed = pl.pallas_call(
        _prep_body,
        out_shape=jax.ShapeDtypeStruct((N, 1), f32),
    )(c0, c1, roles).reshape(N)

    gat = pl.kernel(
        _sc_gather_body,
        out_type=[
            jax.ShapeDtypeStruct((E, EMB), f32),
            jax.ShapeDtypeStruct((E,), f32),
            jax.ShapeDtypeStruct((E,), f32),
        ],
        mesh=mesh,
        scratch_types=[
            pltpu.VMEM((nch, _C), i32),
            pltpu.VMEM((nch, _C), i32),
            pltpu.VMEM((4, _C, EMB), f32),
            pltpu.VMEM((4, _C, EMB), f32),
            pltpu.VMEM((4, _C), f32),
            pltpu.VMEM((4, _C), f32),
            pltpu.VMEM((N,), f32),
        ] + [pltpu.SemaphoreType.DMA] * 8,
        compiler_params=pltpu.CompilerParams(needs_layout_passes=False),
    )
    sim, pkr, pkc = gat(h, packed, row3, col3)
    pkr = pkr.reshape(E, 1)
    pkc = pkc.reshape(E, 1)

    W1 = params['enc_W1']
    w1s = W1[0:EMB]
    w1r = W1[EMB:EMB + 16]
    wd = jnp.zeros((8, HID), f32).at[0].set(W1[EMB + 17]) \
        .at[1].set(W1[EMB + 18])
    rt8 = jnp.zeros((8, 16), f32).at[0:3].set(params['role_table'])
    w1rr = W1[EMB + 19:EMB + 35]
    w1rc = W1[EMB + 35:EMB + 51]
    w1o = jnp.zeros((8, HID), f32).at[0].set(W1[EMB + 16])

    def pad8(v):
        return jnp.zeros((8, v.shape[-1]), f32).at[0].set(v.reshape(-1))

    w3 = jnp.zeros((HID, 8), f32) \
        .at[:, 0].set(params['gate_W'][:, 0]) \
        .at[:, 1].set(params['den_W'][:, 0]) \
        .at[:, 2].set(params['comp_W'][:, 0])
    b3 = jnp.zeros((8, 8), f32) \
        .at[0, 0].set(params['gate_b'][0]) \
        .at[0, 1].set(params['den_b'][0]) \
        .at[0, 2].set(params['comp_b'][0])

    nblk = E // _BE
    full = lambda arr: pl.BlockSpec(arr.shape, lambda i: (0,) * arr.ndim)

    def ebs(w):
        return pl.BlockSpec((_BE, w), lambda i: (i, 0))

    wspecs = []
    wargs = [params['mlp_W0'][0:EMB], params['mlp_W0'][EMB:EMB + 16],
             pad8(params['mlp_b0']),
             w1s, w1r, wd, rt8, w1rr, w1rc, w1o, pad8(params['enc_b1']),
             pad8(params['ln_g']), pad8(params['ln_b']),
             params['enc_W2'], pad8(params['enc_b2']), w3, b3]
    for a in wargs:
        wspecs.append(full(a))

    m_arr, scal, key, stats = pl.pallas_call(
        _pass1_body,
        grid=(nblk,),
        in_specs=[ebs(EMB), ebs(1), ebs(1), ebs(16), ebs(1)] + wspecs,
        out_specs=[ebs(HID), ebs(8), ebs(1),
                   pl.BlockSpec((8, HID), lambda i: (0, 0))],
        out_shape=[jax.ShapeDtypeStruct((E, HID), f32),
                   jax.ShapeDtypeStruct((E, 8), f32),
                   jax.ShapeDtypeStruct((E, 1), u32),
                   jax.ShapeDtypeStruct((8, HID), f32)],
    )(sim, pkr, pkc, rel_embedding, iso, *wargs)

    keyw = key.reshape(E // 128, 128)
    kth = pl.pallas_call(
        _kth_body,
        out_shape=jax.ShapeDtypeStruct((1, 1), u32),
        out_specs=pl.BlockSpec(memory_space=pltpu.SMEM),
    )(keyw)

    mu = stats[0] / float(E)
    ex2 = stats[1] / float(E)
    var = ex2 - mu * mu
    a_vec = params['bn_g'] * lax.rsqrt(var + 1e-5)
    b_vec = params['bn_b'] - mu * a_vec
    bs = params['mlp_bs'].reshape(1, 1)

    weight = pl.pallas_call(
        _pass2_body,
        grid=(nblk,),
        in_specs=[ebs(HID), ebs(8), ebs(1),
                  pl.BlockSpec(memory_space=pltpu.SMEM),
                  pl.BlockSpec(memory_space=pltpu.SMEM),
                  full(jnp.zeros((8, HID))), full(jnp.zeros((8, HID))),
                  pl.BlockSpec((HID, 1), lambda i: (0, 0))],
        out_specs=ebs(1),
        out_shape=jax.ShapeDtypeStruct((E, 1), f32),
    )(m_arr, scal, key, kth, bs, pad8(a_vec), pad8(b_vec),
      params['mlp_Ws'])

    return weight
